# baseline (speedup 1.0000x reference)
import functools

import jax
import jax.numpy as jnp
from jax import lax
from jax.experimental import pallas as pl
from jax.experimental.pallas import tpu as pltpu
from jax.experimental.pallas import tpu_sc as plsc

B = 16384
NUM_FIELDS = 26
VOCAB = 100000
PER_FIELD_DIM = 32
HIDDEN = NUM_FIELDS * PER_FIELD_DIM
OUT_DIM = 128

NC = 2
NS = 16
NW = NC * NS
UNITS_PER_TILE = HIDDEN // NW
QTR_B = B // 4


def _sc_gather(xt2, tables_t):
    mesh = plsc.VectorSubcoreMesh(core_axis_name="c", subcore_axis_name="s")

    @functools.partial(
        pl.kernel,
        mesh=mesh,
        out_type=jax.ShapeDtypeStruct((HIDDEN, B), jnp.float32),
        scratch_types=[
            pltpu.VMEM((VOCAB,), jnp.float32),
            pltpu.VMEM((B,), jnp.int32),
            pltpu.VMEM((QTR_B,), jnp.float32),
            pltpu.VMEM((QTR_B,), jnp.float32),
            pltpu.SemaphoreType.DMA,
            pltpu.SemaphoreType.DMA,
            pltpu.SemaphoreType.DMA,
        ],
        compiler_params=pltpu.CompilerParams(needs_layout_passes=False),
    )
    def k(xt2_hbm, tab_hbm, out_hbm, slice_v, idx_v, ob0, ob1,
          ssem, os0, os1):
        wid = lax.axis_index("s") * NC + lax.axis_index("c")
        obufs = (ob0, ob1)
        osems = (os0, os1)
        writes = [None, None]

        def gather_quarter(ob, q):
            @plsc.parallel_loop(0, QTR_B, 16, unroll=8)
            def gbody(i):
                ids = idx_v[pl.ds(q * QTR_B + i, 16)]
                ob[pl.ds(i, 16)] = plsc.load_gather(slice_v, [ids])

        for u in range(UNITS_PER_TILE):
            g = wid * UNITS_PER_TILE + u
            f = g // PER_FIELD_DIM
            d = g % PER_FIELD_DIM
            slice_cp = pltpu.async_copy(tab_hbm.at[f, d, :], slice_v, ssem)
            if u == 0:
                pltpu.sync_copy(xt2_hbm.at[f, :], idx_v)
            else:
                @pl.when(d == 0)
                def _load_idx():
                    pltpu.sync_copy(xt2_hbm.at[f, :], idx_v)
            slice_cp.wait()
            for q in range(4):
                if writes[q % 2] is not None:
                    writes[q % 2].wait()
                gather_quarter(obufs[q % 2], q)
                writes[q % 2] = pltpu.async_copy(
                    obufs[q % 2],
                    out_hbm.at[g, pl.ds(q * QTR_B, QTR_B)],
                    osems[q % 2])
        writes[0].wait()
        writes[1].wait()

    return k(xt2, tables_t)


def _tc_head(h1t, wt, b2):
    bm = 1024

    def body(e_ref, w_ref, b_ref, o_ref):
        h = jnp.maximum(e_ref[...], 0.0)
        o_ref[...] = (
            jax.lax.dot_general(
                h, w_ref[...],
                dimension_numbers=(((0,), (0,)), ((), ())),
                preferred_element_type=jnp.float32)
            + b_ref[...]
        )

    return pl.pallas_call(
        body,
        grid=(B // bm,),
        in_specs=[
            pl.BlockSpec((HIDDEN, bm), lambda i: (0, i)),
            pl.BlockSpec((HIDDEN, OUT_DIM), lambda i: (0, 0)),
            pl.BlockSpec((1, OUT_DIM), lambda i: (0, 0)),
        ],
        out_specs=pl.BlockSpec((bm, OUT_DIM), lambda i: (i, 0)),
        out_shape=jax.ShapeDtypeStruct((B, OUT_DIM), jnp.float32),
    )(h1t, wt, b2)


def kernel(x, tables, W, b):
    xt2 = jnp.transpose(x)
    tables_t = jnp.transpose(tables, (0, 2, 1))
    h1t = _sc_gather(xt2, tables_t)
    return _tc_head(h1t, W.T, b.reshape(1, OUT_DIM))


# speedup vs baseline: 1.0149x; 1.0149x over previous
import functools

import jax
import jax.numpy as jnp
from jax import lax
from jax.experimental import pallas as pl
from jax.experimental.pallas import tpu as pltpu
from jax.experimental.pallas import tpu_sc as plsc

B = 16384
NUM_FIELDS = 26
VOCAB = 100000
PER_FIELD_DIM = 32
HIDDEN = NUM_FIELDS * PER_FIELD_DIM
OUT_DIM = 128

NC = 2
NS = 16
NW = NC * NS
UNITS_PER_TILE = HIDDEN // NW
QTR_B = B // 4


def _sc_gather(xt2, tables_t):
    mesh = plsc.VectorSubcoreMesh(core_axis_name="c", subcore_axis_name="s")

    @functools.partial(
        pl.kernel,
        mesh=mesh,
        out_type=jax.ShapeDtypeStruct((HIDDEN, B), jnp.float32),
        scratch_types=[
            pltpu.VMEM((VOCAB,), jnp.float32),
            pltpu.VMEM((B,), jnp.int32),
            pltpu.VMEM((QTR_B,), jnp.float32),
            pltpu.VMEM((QTR_B,), jnp.float32),
            pltpu.SemaphoreType.DMA,
            pltpu.SemaphoreType.DMA,
            pltpu.SemaphoreType.DMA,
        ],
        compiler_params=pltpu.CompilerParams(needs_layout_passes=False),
    )
    def k(xt2_hbm, tab_hbm, out_hbm, slice_v, idx_v, ob0, ob1,
          ssem, os0, os1):
        wid = lax.axis_index("s") * NC + lax.axis_index("c")
        obufs = (ob0, ob1)
        osems = (os0, os1)
        writes = [None, None]

        def gather_quarter(ob, q):
            @plsc.parallel_loop(0, QTR_B, 16, unroll=8)
            def gbody(i):
                ids = idx_v[pl.ds(q * QTR_B + i, 16)]
                ob[pl.ds(i, 16)] = plsc.load_gather(slice_v, [ids])

        for u in range(UNITS_PER_TILE):
            g = wid * UNITS_PER_TILE + u
            f = g // PER_FIELD_DIM
            d = g % PER_FIELD_DIM
            slice_cp = pltpu.async_copy(tab_hbm.at[f, d, :], slice_v, ssem)
            if u == 0:
                pltpu.sync_copy(xt2_hbm.at[f, :], idx_v)
            else:
                @pl.when(d == 0)
                def _load_idx():
                    pltpu.sync_copy(xt2_hbm.at[f, :], idx_v)
            slice_cp.wait()
            for q in range(4):
                if writes[q % 2] is not None:
                    writes[q % 2].wait()
                gather_quarter(obufs[q % 2], q)
                writes[q % 2] = pltpu.async_copy(
                    obufs[q % 2],
                    out_hbm.at[g, pl.ds(q * QTR_B, QTR_B)],
                    osems[q % 2])
        writes[0].wait()
        writes[1].wait()

    return k(xt2, tables_t)


def _tc_head(h1t, wt, b2):
    bm = 2048

    def body(e_ref, w_ref, b_ref, o_ref):
        h = jnp.maximum(e_ref[...], 0.0)
        o_ref[...] = (
            jax.lax.dot_general(
                h, w_ref[...],
                dimension_numbers=(((0,), (0,)), ((), ())),
                preferred_element_type=jnp.float32)
            + b_ref[...]
        )

    return pl.pallas_call(
        body,
        grid=(B // bm,),
        in_specs=[
            pl.BlockSpec((HIDDEN, bm), lambda i: (0, i)),
            pl.BlockSpec((HIDDEN, OUT_DIM), lambda i: (0, 0)),
            pl.BlockSpec((1, OUT_DIM), lambda i: (0, 0)),
        ],
        out_specs=pl.BlockSpec((bm, OUT_DIM), lambda i: (i, 0)),
        out_shape=jax.ShapeDtypeStruct((B, OUT_DIM), jnp.float32),
    )(h1t, wt, b2)


def kernel(x, tables, W, b):
    xt2 = jnp.transpose(x)
    tables_t = jnp.transpose(tables, (0, 2, 1))
    h1t = _sc_gather(xt2, tables_t)
    return _tc_head(h1t, W.T, b.reshape(1, OUT_DIM))
